# baseline (speedup 1.0000x reference)
import jax
import jax.numpy as jnp
from jax.experimental import pallas as pl
from jax.experimental.pallas import tpu as pltpu
from jax.experimental.pallas import tpu_sc as plsc


_WB = 128
_LANES = 16


def kernel(inputs, row_embeddings, col_embeddings):
    b, h, w, c = inputs.shape
    wb = _WB
    mesh = plsc.VectorSubcoreMesh(core_axis_name="core", subcore_axis_name="subcore")

    @pl.kernel(
        out_type=jax.ShapeDtypeStruct((b, h, w, c), inputs.dtype),
        mesh=mesh,
        scratch_types=[],
    )
    def sc_kernel(x_hbm, row_hbm, col_hbm, o_hbm):
        def body(x_vmem, row_vmem, col_vmem, o_vmem):
            rvs = [
                row_vmem.at[0, pl.ds(cc, _LANES)][...]
                for cc in range(0, c, _LANES)
            ]

            @plsc.parallel_loop(0, wb, unroll=4)
            def _(wr):
                for k, cc in enumerate(range(0, c, _LANES)):
                    cv = col_vmem.at[wr, pl.ds(cc, _LANES)][...]
                    xv = x_vmem.at[0, 0, wr, pl.ds(cc, _LANES)][...]
                    o_vmem.at[0, 0, wr, pl.ds(cc, _LANES)][...] = xv + rvs[k] + cv

        pltpu.emit_pipeline(
            body,
            grid=(b, h, w // wb),
            in_specs=[
                pl.BlockSpec((1, 1, wb, c), index_map=lambda bi, hi, wi: (bi, hi, wi, 0)),
                pl.BlockSpec((1, c), index_map=lambda bi, hi, wi: (hi, 0)),
                pl.BlockSpec((wb, c), index_map=lambda bi, hi, wi: (wi, 0)),
            ],
            out_specs=[
                pl.BlockSpec((1, 1, wb, c), index_map=lambda bi, hi, wi: (bi, hi, wi, 0)),
            ],
            core_axis_name=("core", "subcore"),
            dimension_semantics=(pltpu.PARALLEL, pltpu.PARALLEL, pltpu.PARALLEL),
        )(x_hbm, row_hbm, col_hbm, o_hbm)

    return sc_kernel(inputs, row_embeddings, col_embeddings)


# speedup vs baseline: 1.2069x; 1.2069x over previous
import jax
import jax.numpy as jnp
from jax.experimental import pallas as pl
from jax.experimental.pallas import tpu as pltpu
from jax.experimental.pallas import tpu_sc as plsc


_KTC = 3
_HB = 8
_DEPTH = 8
_WB = 64
_LANES = 16


def _tc_body(x_hbm, row_ref, col_ref, o_hbm, xb, ob, posb, in_sems, out_sems):
    nb = o_hbm.shape[0]
    h = x_hbm.shape[1]
    hb = posb.shape[0]
    n = (h // hb) * nb
    i = pl.program_id(0)
    hi = i // nb
    bi = i % nb
    slot = jax.lax.rem(i, _DEPTH)

    def read(step):
        s_hi = step // nb
        s_bi = step % nb
        s_slot = jax.lax.rem(step, _DEPTH)
        pltpu.make_async_copy(
            x_hbm.at[s_bi, pl.ds(s_hi * hb, hb)],
            xb.at[s_slot],
            in_sems.at[s_slot],
        ).start()

    @pl.when(i == 0)
    def _prologue():
        for d in range(_DEPTH):
            read(jnp.int32(d))

    pltpu.make_async_copy(
        x_hbm.at[bi, pl.ds(hi * hb, hb)], xb.at[slot], in_sems.at[slot]
    ).wait()

    @pl.when(bi == 0)
    def _pos():
        row = row_ref[pl.ds(hi * hb, hb), :]
        col = col_ref[...]
        posb[...] = row[:, None, :] + col[None, :, :]

    @pl.when(i >= _DEPTH)
    def _drain_out():
        pltpu.make_async_copy(
            ob.at[slot], o_hbm.at[bi, pl.ds(hi * hb, hb)], out_sems.at[slot]
        ).wait()

    ob[slot] = xb[slot] + posb[...]

    pltpu.make_async_copy(
        ob.at[slot], o_hbm.at[bi, pl.ds(hi * hb, hb)], out_sems.at[slot]
    ).start()

    @pl.when(i + _DEPTH < n)
    def _next_read():
        read(i + _DEPTH)

    @pl.when(i == n - 1)
    def _epilogue():
        for d in range(_DEPTH):
            step = n - _DEPTH + d
            s_hi = step // nb
            s_bi = step % nb
            pltpu.make_async_copy(
                ob.at[d], o_hbm.at[s_bi, pl.ds(s_hi * hb, hb)], out_sems.at[d]
            ).wait()


def _tc_part(inputs, row_embeddings, col_embeddings, k):
    b, h, w, c = inputs.shape
    hb = _HB
    n = (h // hb) * k
    return pl.pallas_call(
        _tc_body,
        grid=(n,),
        in_specs=[
            pl.BlockSpec(memory_space=pltpu.MemorySpace.HBM),
            pl.BlockSpec((h, c), lambda i: (0, 0)),
            pl.BlockSpec((w, c), lambda i: (0, 0)),
        ],
        out_specs=pl.BlockSpec(memory_space=pltpu.MemorySpace.HBM),
        out_shape=jax.ShapeDtypeStruct((k, h, w, c), inputs.dtype),
        scratch_shapes=[
            pltpu.VMEM((_DEPTH, hb, w, c), inputs.dtype),
            pltpu.VMEM((_DEPTH, hb, w, c), inputs.dtype),
            pltpu.VMEM((hb, w, c), inputs.dtype),
            pltpu.SemaphoreType.DMA((_DEPTH,)),
            pltpu.SemaphoreType.DMA((_DEPTH,)),
        ],
        compiler_params=pltpu.CompilerParams(
            dimension_semantics=("arbitrary",),
        ),
    )(inputs, row_embeddings, col_embeddings)


def _sc_part(inputs, row_embeddings, col_embeddings, k):
    b, h, w, c = inputs.shape
    wb = _WB
    nb = b - k
    mesh = plsc.VectorSubcoreMesh(core_axis_name="core", subcore_axis_name="subcore")

    @pl.kernel(
        out_type=jax.ShapeDtypeStruct((nb, h, w, c), inputs.dtype),
        mesh=mesh,
        scratch_types=[],
    )
    def sc_kernel(x_hbm, row_hbm, col_hbm, o_hbm):
        def body(x_vmem, row_vmem, col_vmem, o_vmem):
            rvs = [
                row_vmem.at[0, pl.ds(cc, _LANES)][...]
                for cc in range(0, c, _LANES)
            ]

            @plsc.parallel_loop(0, wb, unroll=4)
            def _(wr):
                for j, cc in enumerate(range(0, c, _LANES)):
                    cv = col_vmem.at[wr, pl.ds(cc, _LANES)][...]
                    xv = x_vmem.at[0, 0, wr, pl.ds(cc, _LANES)][...]
                    o_vmem.at[0, 0, wr, pl.ds(cc, _LANES)][...] = xv + rvs[j] + cv

        pltpu.emit_pipeline(
            body,
            grid=(nb, h, w // wb),
            in_specs=[
                pl.BlockSpec(
                    (1, 1, wb, c), index_map=lambda bi, hi, wi: (bi + k, hi, wi, 0)
                ),
                pl.BlockSpec((1, c), index_map=lambda bi, hi, wi: (hi, 0)),
                pl.BlockSpec((wb, c), index_map=lambda bi, hi, wi: (wi, 0)),
            ],
            out_specs=[
                pl.BlockSpec(
                    (1, 1, wb, c), index_map=lambda bi, hi, wi: (bi, hi, wi, 0)
                ),
            ],
            core_axis_name=("core", "subcore"),
            dimension_semantics=(pltpu.PARALLEL, pltpu.PARALLEL, pltpu.PARALLEL),
        )(x_hbm, row_hbm, col_hbm, o_hbm)

    return sc_kernel(inputs, row_embeddings, col_embeddings)


def kernel(inputs, row_embeddings, col_embeddings):
    k = _KTC
    out_tc = _tc_part(inputs, row_embeddings, col_embeddings, k)
    out_sc = _sc_part(inputs, row_embeddings, col_embeddings, k)
    return jnp.concatenate([out_tc, out_sc], axis=0)


# speedup vs baseline: 2.0057x; 1.6619x over previous
import jax
import jax.numpy as jnp
from jax.experimental import pallas as pl
from jax.experimental.pallas import tpu as pltpu
from jax.experimental.pallas import tpu_sc as plsc


_WB = 64
_LANES = 16


def kernel(inputs, row_embeddings, col_embeddings):
    b, h, w, c = inputs.shape
    wb = _WB
    mesh = plsc.VectorSubcoreMesh(core_axis_name="core", subcore_axis_name="subcore")

    @pl.kernel(
        out_type=jax.ShapeDtypeStruct((b, h, w, c), inputs.dtype),
        mesh=mesh,
        scratch_types=[],
    )
    def sc_kernel(x_hbm, row_hbm, col_hbm, o_hbm):
        def body(x_vmem, row_vmem, col_vmem, o_vmem):
            rvs = [
                row_vmem.at[0, pl.ds(cc, _LANES)][...]
                for cc in range(0, c, _LANES)
            ]

            @plsc.parallel_loop(0, wb, unroll=4)
            def _(wr):
                for j, cc in enumerate(range(0, c, _LANES)):
                    cv = col_vmem.at[wr, pl.ds(cc, _LANES)][...]
                    xv = x_vmem.at[0, 0, wr, pl.ds(cc, _LANES)][...]
                    o_vmem.at[0, 0, wr, pl.ds(cc, _LANES)][...] = xv + rvs[j] + cv

        pltpu.emit_pipeline(
            body,
            grid=(b, h, w // wb),
            in_specs=[
                pl.BlockSpec((1, 1, wb, c), index_map=lambda bi, hi, wi: (bi, hi, wi, 0)),
                pl.BlockSpec((1, c), index_map=lambda bi, hi, wi: (hi, 0)),
                pl.BlockSpec((wb, c), index_map=lambda bi, hi, wi: (wi, 0)),
            ],
            out_specs=[
                pl.BlockSpec((1, 1, wb, c), index_map=lambda bi, hi, wi: (bi, hi, wi, 0)),
            ],
            core_axis_name=("core", "subcore"),
            dimension_semantics=(pltpu.PARALLEL, pltpu.PARALLEL, pltpu.PARALLEL),
        )(x_hbm, row_hbm, col_hbm, o_hbm)

    return sc_kernel(inputs, row_embeddings, col_embeddings)
